# Initial kernel scaffold; baseline (speedup 1.0000x reference)
import jax
import jax.numpy as jnp
from jax.experimental import pallas as pl


def kernel(feat, feat_a, adj, graph_neigh, base_w1, spline_w1, base_w2, spline_w2, disc_W, disc_b):
    raise NotImplementedError("write your pallas kernel here")


# speedup vs baseline: 2.9158x; 2.9158x over previous
import numpy as np

import jax
import jax.numpy as jnp
from jax.experimental import pallas as pl

N = 10000
IN_F = 128
OUT_F = 64
G = 5
K = 3
NB = G + K

TM1 = 1000
TM2 = 400
TM3 = 200

_GRID = tuple(
    (np.arange(-K, G + K + 1, dtype=np.float32) * np.float32(2.0 / G)
     - np.float32(1.0)).tolist()
)


def _b_spline_bases(x):
    g = [np.float32(v) for v in _GRID]
    bases = [
        ((x >= g[j]) & (x < g[j + 1])).astype(x.dtype)
        for j in range(len(g) - 1)
    ]
    for p in range(1, K + 1):
        nxt = []
        for j in range(len(bases) - 1):
            left = (x - g[j]) / (g[j + p] - g[j]) * bases[j]
            right = (g[j + p + 1] - x) / (g[j + p + 1] - g[j + 1]) * bases[j + 1]
            nxt.append(left + right)
        bases = nxt
    return bases


def _kan(x, bwt, swt):
    y = jnp.dot(jax.nn.silu(x), bwt, preferred_element_type=jnp.float32)
    for j, b in enumerate(_b_spline_bases(x)):
        y = y + jnp.dot(b, swt[j], preferred_element_type=jnp.float32)
    return y


def _pass1_kernel(feat_ref, feat_a_ref, bw1t_ref, sw1t_ref, zz0_ref):
    bwt = bw1t_ref[...]
    swt = sw1t_ref[...]
    zz0_ref[:, :OUT_F] = _kan(feat_ref[...], bwt, swt)
    zz0_ref[:, OUT_F:] = _kan(feat_a_ref[...], bwt, swt)


def _pass2_kernel(adj_ref, zz0_ref, bw2t_ref, sw2t_ref,
                  z_ref, emb_ref, h0_ref):
    s = jnp.dot(adj_ref[...], zz0_ref[...], preferred_element_type=jnp.float32)
    z = s[:, :OUT_F]
    z_ref[...] = z
    emb_ref[...] = jnp.where(s > 0, s, jnp.exp(jnp.minimum(s, 0.0)) - 1.0)
    h0_ref[...] = _kan(z, bw2t_ref[...], sw2t_ref[...])


def _pass3_kernel(adj_ref, gn_ref, h0_ref, emb_ref, dw_ref, db_ref,
                  h_ref, ret_ref, reta_ref):
    i = pl.program_id(0)
    h_ref[...] = jnp.dot(adj_ref[...], h0_ref[...],
                         preferred_element_type=jnp.float32)
    gn = gn_ref[...]
    r = jnp.dot(gn, emb_ref[...], preferred_element_type=jnp.float32)
    row_sum = jnp.sum(gn, axis=1, keepdims=True)

    ge = r[:, :OUT_F] / row_sum
    ga = r[:, OUT_F:] / row_sum
    nrm_e = jnp.maximum(
        jnp.sqrt(jnp.sum(ge * ge, axis=1, keepdims=True)), 1e-12)
    nrm_a = jnp.maximum(
        jnp.sqrt(jnp.sum(ga * ga, axis=1, keepdims=True)), 1e-12)
    g = jax.nn.sigmoid(ge / nrm_e)
    g_a = jax.nn.sigmoid(ga / nrm_a)

    emb_tile = emb_ref[pl.ds(i * TM3, TM3), :]
    e = emb_tile[:, :OUT_F]
    ea = emb_tile[:, OUT_F:]
    dw = dw_ref[...]
    b = db_ref[0, 0]
    p = jnp.dot(e, dw, preferred_element_type=jnp.float32)
    pa = jnp.dot(ea, dw, preferred_element_type=jnp.float32)
    ret_ref[:, 0:1] = jnp.sum(p * g, axis=1, keepdims=True) + b
    ret_ref[:, 1:2] = jnp.sum(pa * g, axis=1, keepdims=True) + b
    reta_ref[:, 0:1] = jnp.sum(pa * g_a, axis=1, keepdims=True) + b
    reta_ref[:, 1:2] = jnp.sum(p * g_a, axis=1, keepdims=True) + b


def _whole(shape):
    return pl.BlockSpec(shape, lambda i: tuple(0 for _ in shape))


def kernel(feat, feat_a, adj, graph_neigh, base_w1, spline_w1,
           base_w2, spline_w2, disc_W, disc_b):
    f32 = jnp.float32
    bw1t = base_w1.T
    sw1t = jnp.transpose(spline_w1, (2, 1, 0))
    bw2t = base_w2.T
    sw2t = jnp.transpose(spline_w2, (2, 1, 0))
    db = disc_b.reshape(1, 1)

    zz0 = pl.pallas_call(
        _pass1_kernel,
        grid=(N // TM1,),
        in_specs=[
            pl.BlockSpec((TM1, IN_F), lambda i: (i, 0)),
            pl.BlockSpec((TM1, IN_F), lambda i: (i, 0)),
            _whole((IN_F, OUT_F)),
            _whole((NB, IN_F, OUT_F)),
        ],
        out_specs=pl.BlockSpec((TM1, 2 * OUT_F), lambda i: (i, 0)),
        out_shape=jax.ShapeDtypeStruct((N, 2 * OUT_F), f32),
    )(feat, feat_a, bw1t, sw1t)

    z, emb, h0 = pl.pallas_call(
        _pass2_kernel,
        grid=(N // TM2,),
        in_specs=[
            pl.BlockSpec((TM2, N), lambda i: (i, 0)),
            _whole((N, 2 * OUT_F)),
            _whole((OUT_F, IN_F)),
            _whole((NB, OUT_F, IN_F)),
        ],
        out_specs=[
            pl.BlockSpec((TM2, OUT_F), lambda i: (i, 0)),
            pl.BlockSpec((TM2, 2 * OUT_F), lambda i: (i, 0)),
            pl.BlockSpec((TM2, IN_F), lambda i: (i, 0)),
        ],
        out_shape=[
            jax.ShapeDtypeStruct((N, OUT_F), f32),
            jax.ShapeDtypeStruct((N, 2 * OUT_F), f32),
            jax.ShapeDtypeStruct((N, IN_F), f32),
        ],
    )(adj, zz0, bw2t, sw2t)

    h, ret, ret_a = pl.pallas_call(
        _pass3_kernel,
        grid=(N // TM3,),
        in_specs=[
            pl.BlockSpec((TM3, N), lambda i: (i, 0)),
            pl.BlockSpec((TM3, N), lambda i: (i, 0)),
            _whole((N, IN_F)),
            _whole((N, 2 * OUT_F)),
            _whole((OUT_F, OUT_F)),
            _whole((1, 1)),
        ],
        out_specs=[
            pl.BlockSpec((TM3, IN_F), lambda i: (i, 0)),
            pl.BlockSpec((TM3, 2), lambda i: (i, 0)),
            pl.BlockSpec((TM3, 2), lambda i: (i, 0)),
        ],
        out_shape=[
            jax.ShapeDtypeStruct((N, IN_F), f32),
            jax.ShapeDtypeStruct((N, 2), f32),
            jax.ShapeDtypeStruct((N, 2), f32),
        ],
    )(adj, graph_neigh, h0, emb, disc_W, db)

    return (z, h, ret, ret_a)


# speedup vs baseline: 3.1208x; 1.0703x over previous
import numpy as np

import jax
import jax.numpy as jnp
from jax.experimental import pallas as pl

N = 10000
IN_F = 128
OUT_F = 64
G = 5
K = 3
NB = G + K

TM1 = 1000
TM2 = 400
TM3 = 200

_GRID = tuple(
    (np.arange(-K, G + K + 1, dtype=np.float32) * np.float32(2.0 / G)
     - np.float32(1.0)).tolist()
)


def _b_spline_bases(x):
    g = [np.float32(v) for v in _GRID]
    inv_h = np.float32(1.0) / (g[1] - g[0])
    u = (x - g[0]) * inv_h
    cf = jnp.floor(u)
    t = u - cf
    t2 = t * t
    t3 = t2 * t
    c16 = np.float32(1.0 / 6.0)
    c12 = np.float32(0.5)
    s0 = t3 * c16
    s1 = ((-c12 * t + c12) * t + c12) * t + c16
    s2 = (c12 * t - np.float32(1.0)) * t2 + np.float32(4.0 / 6.0)
    s3 = ((-c16 * t + c12) * t - c12) * t + c16
    seg = (s0, s1, s2, s3)
    bases = []
    for j in range(NB):
        b = jnp.zeros_like(x)
        for m in range(4):
            b = b + jnp.where(cf == np.float32(j + m), seg[m], np.float32(0.0))
        bases.append(b)
    return bases


def _kan(x, bwt, swt):
    y = jnp.dot(jax.nn.silu(x), bwt, preferred_element_type=jnp.float32)
    for j, b in enumerate(_b_spline_bases(x)):
        y = y + jnp.dot(b, swt[j], preferred_element_type=jnp.float32)
    return y


def _pass1_kernel(feat_ref, feat_a_ref, bw1t_ref, sw1t_ref, zz0_ref):
    bwt = bw1t_ref[...]
    swt = sw1t_ref[...]
    zz0_ref[:, :OUT_F] = _kan(feat_ref[...], bwt, swt)
    zz0_ref[:, OUT_F:] = _kan(feat_a_ref[...], bwt, swt)


def _pass2_kernel(adj_ref, zz0_ref, bw2t_ref, sw2t_ref,
                  z_ref, emb_ref, h0_ref):
    s = jnp.dot(adj_ref[...], zz0_ref[...], preferred_element_type=jnp.float32)
    z = s[:, :OUT_F]
    z_ref[...] = z
    emb_ref[...] = jnp.where(s > 0, s, jnp.exp(jnp.minimum(s, 0.0)) - 1.0)
    h0_ref[...] = _kan(z, bw2t_ref[...], sw2t_ref[...])


def _pass3_kernel(adj_ref, gn_ref, h0_ref, emb_ref, dw_ref, db_ref,
                  h_ref, ret_ref, reta_ref):
    i = pl.program_id(0)
    h_ref[...] = jnp.dot(adj_ref[...], h0_ref[...],
                         preferred_element_type=jnp.float32)
    gn = gn_ref[...]
    r = jnp.dot(gn, emb_ref[...], preferred_element_type=jnp.float32)
    row_sum = jnp.sum(gn, axis=1, keepdims=True)

    ge = r[:, :OUT_F] / row_sum
    ga = r[:, OUT_F:] / row_sum
    nrm_e = jnp.maximum(
        jnp.sqrt(jnp.sum(ge * ge, axis=1, keepdims=True)), 1e-12)
    nrm_a = jnp.maximum(
        jnp.sqrt(jnp.sum(ga * ga, axis=1, keepdims=True)), 1e-12)
    g = jax.nn.sigmoid(ge / nrm_e)
    g_a = jax.nn.sigmoid(ga / nrm_a)

    emb_tile = emb_ref[pl.ds(i * TM3, TM3), :]
    e = emb_tile[:, :OUT_F]
    ea = emb_tile[:, OUT_F:]
    dw = dw_ref[...]
    b = db_ref[0, 0]
    p = jnp.dot(e, dw, preferred_element_type=jnp.float32)
    pa = jnp.dot(ea, dw, preferred_element_type=jnp.float32)
    ret_ref[:, 0:1] = jnp.sum(p * g, axis=1, keepdims=True) + b
    ret_ref[:, 1:2] = jnp.sum(pa * g, axis=1, keepdims=True) + b
    reta_ref[:, 0:1] = jnp.sum(pa * g_a, axis=1, keepdims=True) + b
    reta_ref[:, 1:2] = jnp.sum(p * g_a, axis=1, keepdims=True) + b


def _whole(shape):
    return pl.BlockSpec(shape, lambda i: tuple(0 for _ in shape))


def kernel(feat, feat_a, adj, graph_neigh, base_w1, spline_w1,
           base_w2, spline_w2, disc_W, disc_b):
    f32 = jnp.float32
    bw1t = base_w1.T
    sw1t = jnp.transpose(spline_w1, (2, 1, 0))
    bw2t = base_w2.T
    sw2t = jnp.transpose(spline_w2, (2, 1, 0))
    db = disc_b.reshape(1, 1)

    zz0 = pl.pallas_call(
        _pass1_kernel,
        grid=(N // TM1,),
        in_specs=[
            pl.BlockSpec((TM1, IN_F), lambda i: (i, 0)),
            pl.BlockSpec((TM1, IN_F), lambda i: (i, 0)),
            _whole((IN_F, OUT_F)),
            _whole((NB, IN_F, OUT_F)),
        ],
        out_specs=pl.BlockSpec((TM1, 2 * OUT_F), lambda i: (i, 0)),
        out_shape=jax.ShapeDtypeStruct((N, 2 * OUT_F), f32),
    )(feat, feat_a, bw1t, sw1t)

    z, emb, h0 = pl.pallas_call(
        _pass2_kernel,
        grid=(N // TM2,),
        in_specs=[
            pl.BlockSpec((TM2, N), lambda i: (i, 0)),
            _whole((N, 2 * OUT_F)),
            _whole((OUT_F, IN_F)),
            _whole((NB, OUT_F, IN_F)),
        ],
        out_specs=[
            pl.BlockSpec((TM2, OUT_F), lambda i: (i, 0)),
            pl.BlockSpec((TM2, 2 * OUT_F), lambda i: (i, 0)),
            pl.BlockSpec((TM2, IN_F), lambda i: (i, 0)),
        ],
        out_shape=[
            jax.ShapeDtypeStruct((N, OUT_F), f32),
            jax.ShapeDtypeStruct((N, 2 * OUT_F), f32),
            jax.ShapeDtypeStruct((N, IN_F), f32),
        ],
    )(adj, zz0, bw2t, sw2t)

    h, ret, ret_a = pl.pallas_call(
        _pass3_kernel,
        grid=(N // TM3,),
        in_specs=[
            pl.BlockSpec((TM3, N), lambda i: (i, 0)),
            pl.BlockSpec((TM3, N), lambda i: (i, 0)),
            _whole((N, IN_F)),
            _whole((N, 2 * OUT_F)),
            _whole((OUT_F, OUT_F)),
            _whole((1, 1)),
        ],
        out_specs=[
            pl.BlockSpec((TM3, IN_F), lambda i: (i, 0)),
            pl.BlockSpec((TM3, 2), lambda i: (i, 0)),
            pl.BlockSpec((TM3, 2), lambda i: (i, 0)),
        ],
        out_shape=[
            jax.ShapeDtypeStruct((N, IN_F), f32),
            jax.ShapeDtypeStruct((N, 2), f32),
            jax.ShapeDtypeStruct((N, 2), f32),
        ],
    )(adj, graph_neigh, h0, emb, disc_W, db)

    return (z, h, ret, ret_a)
